# Initial kernel scaffold; baseline (speedup 1.0000x reference)
import jax
import jax.numpy as jnp
from jax.experimental import pallas as pl


def kernel(label_probs, embedding_table):
    raise NotImplementedError("write your pallas kernel here")


# speedup vs baseline: 10.0293x; 10.0293x over previous
import jax
import jax.numpy as jnp
from jax.experimental import pallas as pl

N = 100000
D = 128
BLOCK = 2000


def _body(p_ref, t_ref, o_ref):
    p = p_ref[0, 0, :]
    row0 = t_ref[0, :]
    diff = t_ref[1, :]
    o_ref[...] = row0[None, :] + p[:, None] * diff[None, :]


def kernel(label_probs, embedding_table):
    params = jnp.stack(
        [embedding_table[0], embedding_table[1] - embedding_table[0]], axis=0
    )
    num_blocks = N // BLOCK
    p3 = label_probs.reshape(num_blocks, 1, BLOCK)
    return pl.pallas_call(
        _body,
        grid=(num_blocks,),
        in_specs=[
            pl.BlockSpec((1, 1, BLOCK), lambda g: (g, 0, 0)),
            pl.BlockSpec((2, D), lambda g: (0, 0)),
        ],
        out_specs=pl.BlockSpec((BLOCK, D), lambda g: (g, 0)),
        out_shape=jax.ShapeDtypeStruct((N, D), jnp.float32),
    )(p3, params)


# speedup vs baseline: 10.4132x; 1.0383x over previous
import functools
import jax
import jax.numpy as jnp
from jax import lax
from jax.experimental import pallas as pl
from jax.experimental.pallas import tpu as pltpu
from jax.experimental.pallas import tpu_sc as plsc

N = 100000
D = 128
NWORKER = 32
R = 128
NCHUNK = 25
ROWS_PER_W = R * NCHUNK
NPAD = NWORKER * ROWS_PER_W
TAIL = N % R
TAIL_S = (N % ROWS_PER_W) // R


def _sc_body(p3_hbm, t_hbm, out_hbm, tbuf, pbuf, obuf0, obuf1, sem0, sem1):
    w = lax.axis_index("c") * 16 + lax.axis_index("s")
    pltpu.sync_copy(t_hbm, tbuf)
    pltpu.sync_copy(p3_hbm.at[w], pbuf)

    r0 = [tbuf[0, pl.ds(j * 16, 16)] for j in range(8)]
    df = [tbuf[1, pl.ds(j * 16, 16)] - r0[j] for j in range(8)]
    bidx = [jnp.full((16,), r, jnp.int32) for r in range(16)]

    obufs = (obuf0, obuf1)
    sems = (sem0, sem1)
    base = w * ROWS_PER_W

    def compute_chunk(s, b):
        def grp(k, _):
            v16 = pbuf[s, pl.ds(k * 16, 16)]
            for r in range(16):
                pv = v16[bidx[r]]
                for j in range(8):
                    obufs[b][k * 16 + r, pl.ds(j * 16, 16)] = (
                        r0[j] + pv * df[j])
            return 0
        return grp

    def pair(sp, _):
        for b in (0, 1):
            s = 2 * sp + b
            g0 = base + s * R
            full = jnp.logical_and(g0 + R <= N, s < NCHUNK)

            @pl.when(jnp.logical_and(full, s >= 2))
            def _():
                pltpu.make_async_copy(
                    obufs[b], out_hbm.at[pl.ds(0, R)], sems[b]).wait()

            @pl.when(full)
            def _():
                lax.fori_loop(0, R // 16, compute_chunk(s, b), 0)
                pltpu.async_copy(obufs[b], out_hbm.at[pl.ds(g0, R)],
                                 sems[b])
        return 0

    lax.fori_loop(0, (NCHUNK + 1) // 2, pair, 0)

    @pl.when(jnp.logical_and(base < N, base + ROWS_PER_W > N))
    def _():
        pltpu.make_async_copy(
            obufs[0], out_hbm.at[pl.ds(0, R)], sems[0]).wait()
        lax.fori_loop(0, TAIL // 16, compute_chunk(TAIL_S, 0), 0)
        pltpu.sync_copy(obufs[0].at[pl.ds(0, TAIL)],
                        out_hbm.at[pl.ds(base + TAIL_S * R, TAIL)])

    pltpu.make_async_copy(obufs[1], out_hbm.at[pl.ds(0, R)], sems[1]).wait()

    @pl.when(base + ROWS_PER_W <= N)
    def _():
        pltpu.make_async_copy(
            obufs[0], out_hbm.at[pl.ds(0, R)], sems[0]).wait()


def kernel(label_probs, embedding_table):
    mesh = plsc.VectorSubcoreMesh(core_axis_name="c", subcore_axis_name="s")
    k = functools.partial(
        pl.kernel,
        mesh=mesh,
        out_type=jax.ShapeDtypeStruct((N, D), jnp.float32),
        scratch_types=[
            pltpu.VMEM((3, D), jnp.float32),
            pltpu.VMEM((NCHUNK, R), jnp.float32),
            pltpu.VMEM((R, D), jnp.float32),
            pltpu.VMEM((R, D), jnp.float32),
            pltpu.SemaphoreType.DMA,
            pltpu.SemaphoreType.DMA,
        ],
    )(_sc_body)
    p3 = jnp.pad(label_probs, (0, NPAD - N)).reshape(NWORKER, NCHUNK, R)
    return k(p3, embedding_table)
